# Initial kernel scaffold; baseline (speedup 1.0000x reference)
import jax
import jax.numpy as jnp
from jax.experimental import pallas as pl


def kernel(scores, embeddings, target_idx, applicable_mask):
    raise NotImplementedError("write your pallas kernel here")


# speedup vs baseline: 2.3253x; 2.3253x over previous
import functools

import jax
import jax.numpy as jnp
from jax import lax
from jax.experimental import pallas as pl
from jax.experimental.pallas import tpu as pltpu
from jax.experimental.pallas import tpu_sc as plsc

_TEMP = 0.07
_MARGIN = 1.0
_ALPHA = 0.3
_NEG_INF = float("-inf")
_INT_MIN = -2147483648

_NS = 16
_L = 16
_CH = 8192

_W_ROUND = 0
_W_M = 32
_W_THI = 33
_W_TLO = 34
_W_MT = 35
_W_CM1 = 36
_W_CGT = 37
_MB_SMAX = 40
_MB_MAXI = 72
_MB_LI = 104
_MB_SEXP = 136
_MB_SM1 = 168
_MB_SGT = 200
_NWORDS = 240


def _xlane(v, op):
    idx = lax.iota(jnp.int32, _L)
    for sh in (1, 2, 4, 8):
        v = op(v, v.at[idx ^ sh].get(mode="promise_in_bounds"))
    return v[0]


def _to_skey(s):
    u = lax.bitcast_convert_type(s, jnp.int32)
    return jnp.where(u >= 0, u, jnp.int32(_INT_MIN) - u)


def _from_skey(key):
    u = jnp.where(key >= 0, key, jnp.int32(_INT_MIN) - key)
    return lax.bitcast_convert_type(u, jnp.float32)


def _halves(x_f32):
    b = lax.bitcast_convert_type(x_f32, jnp.int32)
    hi = lax.shift_right_logical(b, 16)
    lo = b & jnp.int32(0xFFFF)
    return hi, lo


def _sc_body(scores_hbm, mask_hbm, ti_hbm, outf_hbm, outi_hbm,
             a_ref, sbuf, mbuf, tibuf, obuf_f, obuf_i, sm,
             *, l_sub, n_real):
    nch = l_sub // _CH
    nstep = _CH // _L
    nvec = l_sub // _L
    wid = lax.axis_index("s")
    base = wid * l_sub
    neg_inf = jnp.float32(_NEG_INF)
    lanes = lax.iota(jnp.int32, _L)
    zero = jnp.float32(0.0)
    one_i = jnp.int32(1)

    pltpu.sync_copy(ti_hbm, tibuf)
    ti = tibuf[...][0]

    def _z(j, x):
        sm[j] = jnp.int32(0)
        return x

    lax.fori_loop(0, _NWORDS, _z, jnp.int32(0))
    plsc.subcore_barrier()

    def _poll(word, expect):
        def cond(v):
            return (v & jnp.int32(31)) != expect

        def body(v):
            return plsc.fetch_and_add(sm.at[word], jnp.int32(0),
                                      subcore_id=wid)

        v = lax.while_loop(cond, body, jnp.int32(0))
        return lax.shift_right_logical(v, 5)

    def chunk_body(c, carry):
        cnt, mxm, mxi, tac, mta, mxn = carry
        off = base + c * _CH
        pltpu.sync_copy(scores_hbm.at[pl.ds(off, _CH)], sbuf)
        pltpu.sync_copy(mask_hbm.at[pl.ds(off, _CH)], mbuf)

        def step(j, carry2):
            cnt, mxm, mxi, tac, mta, mxn = carry2
            p = j * _L
            s = sbuf[pl.ds(p, _L)]
            mk = mbuf[pl.ds(p, _L)]
            mkb = mk != 0
            gidx = off + p + lanes
            is_t = gidx == ti
            cnt = cnt + jnp.where(mkb, 1, 0).astype(jnp.int32)
            mxm = jnp.maximum(mxm, jnp.where(mkb, s, neg_inf))
            mxi = jnp.maximum(mxi, jnp.where(mkb, neg_inf, s))
            tac = jnp.maximum(tac, jnp.where(is_t, s, neg_inf))
            mta = jnp.maximum(mta, jnp.where(is_t, mk.astype(jnp.float32),
                                             neg_inf))
            negb = mkb & jnp.logical_not(is_t)
            mxn = jnp.maximum(mxn, jnp.where(negb, s, neg_inf))
            sk = _to_skey(s)
            a_ref[pl.ds(c * _CH + p, _L)] = jnp.where(negb, sk,
                                                      jnp.int32(_INT_MIN))
            return cnt, mxm, mxi, tac, mta, mxn

        return lax.fori_loop(0, nstep, step, carry)

    finit = jnp.full((_L,), neg_inf, jnp.float32)
    cnt, mxm, mxi, tac, mta, mxn = lax.fori_loop(
        0, nch, chunk_body,
        (jnp.zeros((_L,), jnp.int32), finit, finit, finit, finit, finit))

    m_loc = _xlane(cnt.astype(jnp.float32), jnp.add).astype(jnp.int32)
    smax_loc = _xlane(mxm, jnp.maximum)
    maxi_loc = _xlane(mxi, jnp.maximum)
    t_loc = _xlane(tac, jnp.maximum)
    mt_loc = _xlane(mta, jnp.maximum)
    li_loc = _xlane(mxn, jnp.maximum)

    inv_t = jnp.float32(1.0 / _TEMP)
    li_t = li_loc * inv_t

    def p2_step(j, sexp):
        sk = a_ref[pl.ds(j * _L, _L)]
        valid = sk != jnp.int32(_INT_MIN)
        s = _from_skey(sk)
        e = jnp.exp(s * inv_t - li_t)
        return sexp + jnp.where(valid, e, zero)

    sexp_loc = _xlane(
        lax.fori_loop(0, nvec, p2_step, jnp.zeros((_L,), jnp.float32)),
        jnp.add)
    sexp_loc = jnp.where(li_loc == neg_inf, zero, sexp_loc)

    owner = (ti >= base) & (ti < base + l_sub)
    thi, tlo = _halves(t_loc)
    smax_hi, smax_lo = _halves(smax_loc)
    maxi_hi, maxi_lo = _halves(maxi_loc)
    li_hi, li_lo = _halves(li_loc)
    se_hi, se_lo = _halves(sexp_loc)
    mt_int = jnp.where(mt_loc > jnp.float32(0.5), one_i, jnp.int32(0))
    zi = jnp.int32(0)

    def m1_send(j, x):
        tgt = j & jnp.int32(15)
        ws = lax.shift_right_logical(j, 4)
        word = jnp.where(ws == 0, _W_M, zi)
        val = jnp.where(ws == 0, m_loc, zi)
        word = jnp.where(ws == 1, _W_THI, word)
        val = jnp.where(ws == 1, jnp.where(owner, thi, zi), val)
        word = jnp.where(ws == 2, _W_TLO, word)
        val = jnp.where(ws == 2, jnp.where(owner, tlo, zi), val)
        word = jnp.where(ws == 3, _W_MT, word)
        val = jnp.where(ws == 3, jnp.where(owner, mt_int, zi), val)
        mb = ws - 4
        mtyp = lax.shift_right_logical(mb, 1)
        half = mb & one_i
        mb_base = jnp.where(mtyp == 0, _MB_SMAX, zi)
        mb_val_h = jnp.where(mtyp == 0, smax_hi, zi)
        mb_val_l = jnp.where(mtyp == 0, smax_lo, zi)
        mb_base = jnp.where(mtyp == 1, _MB_MAXI, mb_base)
        mb_val_h = jnp.where(mtyp == 1, maxi_hi, mb_val_h)
        mb_val_l = jnp.where(mtyp == 1, maxi_lo, mb_val_l)
        mb_base = jnp.where(mtyp == 2, _MB_LI, mb_base)
        mb_val_h = jnp.where(mtyp == 2, li_hi, mb_val_h)
        mb_val_l = jnp.where(mtyp == 2, li_lo, mb_val_l)
        mb_base = jnp.where(mtyp == 3, _MB_SEXP, mb_base)
        mb_val_h = jnp.where(mtyp == 3, se_hi, mb_val_h)
        mb_val_l = jnp.where(mtyp == 3, se_lo, mb_val_l)
        word = jnp.where(ws >= 4, mb_base + wid * 2 + half, word)
        val = jnp.where(ws >= 4, jnp.where(half == 0, mb_val_h, mb_val_l),
                        val)
        plsc.fetch_and_add(sm.at[word], val * 32 + 1, subcore_id=tgt)
        return x

    lax.fori_loop(0, 12 * _NS, m1_send, jnp.int32(0))

    def m1_poll(j, carry):
        sh_vec, hi0, lo0, hi1, lo1, hi2, lo2, hi3, lo3 = carry
        is_sh = j < 4
        writer = lax.shift_right_logical(j - 4, 3) & jnp.int32(15)
        rem = (j - 4) & jnp.int32(7)
        mtyp = lax.shift_right_logical(rem, 1)
        half = rem & one_i
        word = jnp.where(is_sh, _W_M + j, zi)
        mb_base = jnp.where(mtyp == 0, _MB_SMAX, zi)
        mb_base = jnp.where(mtyp == 1, _MB_MAXI, mb_base)
        mb_base = jnp.where(mtyp == 2, _MB_LI, mb_base)
        mb_base = jnp.where(mtyp == 3, _MB_SEXP, mb_base)
        word = jnp.where(is_sh, word, mb_base + writer * 2 + half)
        expect = jnp.where(is_sh, jnp.int32(_NS), one_i)
        val = _poll(word, expect)
        sh_vec = jnp.where(is_sh & (lanes == j), val, sh_vec)
        upd = jnp.logical_not(is_sh) & (lanes == writer)
        hi0 = jnp.where(upd & (mtyp == 0) & (half == 0), val, hi0)
        lo0 = jnp.where(upd & (mtyp == 0) & (half == 1), val, lo0)
        hi1 = jnp.where(upd & (mtyp == 1) & (half == 0), val, hi1)
        lo1 = jnp.where(upd & (mtyp == 1) & (half == 1), val, lo1)
        hi2 = jnp.where(upd & (mtyp == 2) & (half == 0), val, hi2)
        lo2 = jnp.where(upd & (mtyp == 2) & (half == 1), val, lo2)
        hi3 = jnp.where(upd & (mtyp == 3) & (half == 0), val, hi3)
        lo3 = jnp.where(upd & (mtyp == 3) & (half == 1), val, lo3)
        return sh_vec, hi0, lo0, hi1, lo1, hi2, lo2, hi3, lo3

    zv = jnp.zeros((_L,), jnp.int32)
    (sh_vec, hi0, lo0, hi1, lo1, hi2, lo2, hi3, lo3) = lax.fori_loop(
        0, 4 + 4 * _NS * 2, m1_poll, (zv,) * 9)

    def _vec_f32(hi, lo):
        return lax.bitcast_convert_type(
            lax.shift_left(hi, 16) | lo, jnp.float32)

    smax_vec = _vec_f32(hi0, lo0)
    maxi_vec = _vec_f32(hi1, lo1)
    li_vec = _vec_f32(hi2, lo2)
    sexp_vec = _vec_f32(hi3, lo3)

    m_g = sh_vec[0].astype(jnp.float32)
    t_bits = lax.shift_left(sh_vec[1], 16) | sh_vec[2]
    t_g = lax.bitcast_convert_type(t_bits, jnp.float32)
    maskt_g = sh_vec[3].astype(jnp.float32)
    smax_g = _xlane(smax_vec, jnp.maximum)
    maxinap_g = _xlane(maxi_vec, jnp.maximum)
    scale = jnp.exp((li_vec - smax_g) * inv_t)
    sumexp_neg = _xlane(jnp.where(li_vec == neg_inf, zero, sexp_vec * scale),
                        jnp.add)

    mi = m_g.astype(jnp.int32)
    k_i = jnp.maximum(one_i, (mi - 1) >> 1)

    def round_body(r, carry3):
        p_o = carry3
        b = 31 - r
        cand = p_o + jnp.left_shift(jnp.uint32(1), b.astype(jnp.uint32))
        cand_sk = lax.bitcast_convert_type(cand ^ jnp.uint32(0x80000000),
                                           jnp.int32)

        def cstep(j, ccnt):
            sk = a_ref[pl.ds(j * _L, _L)]
            return ccnt + jnp.where(sk >= cand_sk, 1, 0).astype(jnp.int32)

        ccnt = lax.fori_loop(0, nvec, cstep, jnp.zeros((_L,), jnp.int32))
        c_loc = _xlane(ccnt.astype(jnp.float32), jnp.add).astype(jnp.int32)

        def r_send(j, x):
            plsc.fetch_and_add(sm.at[_W_ROUND + r], c_loc * 32 + 1,
                               subcore_id=j)
            return x

        lax.fori_loop(0, _NS, r_send, jnp.int32(0))
        c_g = _poll(_W_ROUND + r, jnp.int32(_NS))
        take = c_g >= k_i
        return jnp.where(take, cand, p_o)

    p_o = lax.fori_loop(0, 32, round_body, jnp.uint32(0))
    theta_key = lax.bitcast_convert_type(p_o ^ jnp.uint32(0x80000000),
                                         jnp.int32)

    t1key = _to_skey(jnp.broadcast_to(t_g - jnp.float32(_MARGIN), (_L,)))[0]

    def p4_step(j, carry4):
        cm1, sm1, cgt, sgt = carry4
        sk = a_ref[pl.ds(j * _L, _L)]
        valid = sk != jnp.int32(_INT_MIN)
        s = _from_skey(sk)
        gt1 = valid & (sk > t1key)
        gth = valid & (sk > theta_key)
        cm1 = cm1 + jnp.where(gt1, 1, 0).astype(jnp.int32)
        sm1 = sm1 + jnp.where(gt1, s, zero)
        cgt = cgt + jnp.where(gth, 1, 0).astype(jnp.int32)
        sgt = sgt + jnp.where(gth, s, zero)
        return cm1, sm1, cgt, sgt

    cm1, sm1, cgt, sgt = lax.fori_loop(
        0, nvec, p4_step,
        (zv, jnp.zeros((_L,), jnp.float32), zv,
         jnp.zeros((_L,), jnp.float32)))
    cm1_loc = _xlane(cm1.astype(jnp.float32), jnp.add).astype(jnp.int32)
    sm1_loc = _xlane(sm1, jnp.add)
    cgt_loc = _xlane(cgt.astype(jnp.float32), jnp.add).astype(jnp.int32)
    sgt_loc = _xlane(sgt, jnp.add)
    sm1_hi, sm1_lo = _halves(sm1_loc)
    sgt_hi, sgt_lo = _halves(sgt_loc)

    def f_send(j, x):
        tgt = j & jnp.int32(15)
        ws = lax.shift_right_logical(j, 4)
        word = jnp.where(ws == 0, _W_CM1, zi)
        val = jnp.where(ws == 0, cm1_loc, zi)
        word = jnp.where(ws == 1, _W_CGT, word)
        val = jnp.where(ws == 1, cgt_loc, val)
        word = jnp.where(ws == 2, _MB_SM1 + wid * 2, word)
        val = jnp.where(ws == 2, sm1_hi, val)
        word = jnp.where(ws == 3, _MB_SM1 + wid * 2 + 1, word)
        val = jnp.where(ws == 3, sm1_lo, val)
        word = jnp.where(ws == 4, _MB_SGT + wid * 2, word)
        val = jnp.where(ws == 4, sgt_hi, val)
        word = jnp.where(ws == 5, _MB_SGT + wid * 2 + 1, word)
        val = jnp.where(ws == 5, sgt_lo, val)
        plsc.fetch_and_add(sm.at[word], val * 32 + 1, subcore_id=tgt)
        return x

    lax.fori_loop(0, 6 * _NS, f_send, jnp.int32(0))

    def f_poll(j, carry):
        sh_vec, hiA, loA, hiB, loB = carry
        is_sh = j < 2
        writer = lax.shift_right_logical(j - 2, 2) & jnp.int32(15)
        rem = (j - 2) & jnp.int32(3)
        mtyp = lax.shift_right_logical(rem, 1)
        half = rem & one_i
        word = jnp.where(is_sh, _W_CM1 + j, zi)
        mb_base = jnp.where(mtyp == 0, _MB_SM1, jnp.int32(_MB_SGT))
        word = jnp.where(is_sh, word, mb_base + writer * 2 + half)
        expect = jnp.where(is_sh, jnp.int32(_NS), one_i)
        val = _poll(word, expect)
        sh_vec = jnp.where(is_sh & (lanes == j), val, sh_vec)
        upd = jnp.logical_not(is_sh) & (lanes == writer)
        hiA = jnp.where(upd & (mtyp == 0) & (half == 0), val, hiA)
        loA = jnp.where(upd & (mtyp == 0) & (half == 1), val, loA)
        hiB = jnp.where(upd & (mtyp == 1) & (half == 0), val, hiB)
        loB = jnp.where(upd & (mtyp == 1) & (half == 1), val, loB)
        return sh_vec, hiA, loA, hiB, loB

    (fsh, hiA, loA, hiB, loB) = lax.fori_loop(
        0, 2 + 2 * _NS * 2, f_poll, (zv,) * 5)
    cnt_m1 = fsh[0].astype(jnp.float32)
    cnt_gt = fsh[1].astype(jnp.float32)
    sum_m1 = _xlane(_vec_f32(hiA, loA), jnp.add)
    sum_gt = _xlane(_vec_f32(hiB, loB), jnp.add)

    @pl.when(wid == 0)
    def _():
        ov = jnp.zeros((_L,), jnp.float32)
        ov = jnp.where(lanes == 0, m_g, ov)
        ov = jnp.where(lanes == 1, smax_g, ov)
        ov = jnp.where(lanes == 2, maxinap_g, ov)
        ov = jnp.where(lanes == 3, t_g, ov)
        ov = jnp.where(lanes == 4, maskt_g, ov)
        ov = jnp.where(lanes == 5, sumexp_neg, ov)
        ov = jnp.where(lanes == 6, cnt_m1, ov)
        ov = jnp.where(lanes == 7, sum_m1, ov)
        ov = jnp.where(lanes == 8, cnt_gt, ov)
        ov = jnp.where(lanes == 9, sum_gt, ov)
        obuf_f[...] = ov
        obuf_i[...] = jnp.where(lanes == 0, theta_key, jnp.int32(0))
        pltpu.sync_copy(obuf_f, outf_hbm)
        pltpu.sync_copy(obuf_i, outi_hbm)


def _finish_body(f_ref, i_ref, ti_ref, out_ref, *, n_real):
    def g(i):
        return f_ref[0:1, i:i+1]

    m = g(0)
    smax = g(1)
    maxinap = g(2)
    t = g(3)
    mask_t = g(4)
    sumexp_neg = g(5)
    cnt_m1 = g(6)
    sum_m1 = g(7)
    cnt_gt = g(8)
    sum_gt = g(9)
    key = i_ref[0:1, 0:1]
    ti_raw = ti_ref[0:1, 0:1]

    temp = jnp.float32(_TEMP)
    one = jnp.float32(1.0)
    big_m = smax / temp
    lt = t / temp
    et = jnp.exp(lt - big_m)
    sumexp = sumexp_neg + jnp.where(mask_t > 0.5, et, 0.0)
    lse = jnp.log(sumexp) + big_m
    li = lse - lt
    p_t = et / sumexp
    fw = (one - p_t) * (one - p_t)
    li = fw * li

    mi = m.astype(jnp.int32)
    k_i = jnp.maximum(jnp.int32(1),
                      lax.shift_right_arithmetic(mi - 1, jnp.int32(1)))
    kf = k_i.astype(jnp.float32)

    u = jnp.where(key >= 0, key, jnp.int32(_INT_MIN) - key)
    theta = lax.bitcast_convert_type(u, jnp.float32)

    s_a = cnt_m1 * (one - t) + sum_m1
    s_b = cnt_gt * (one - t) + sum_gt + (kf - cnt_gt) * (one + theta - t)
    s_sel = jnp.where(cnt_m1 >= kf, s_b, s_a)
    hard = s_sel / kf

    bl = jnp.where(m < jnp.float32(n_real),
                   jnp.maximum(one + maxinap - t, 0.0), 0.0)

    total = li + jnp.float32(0.5) * hard + jnp.float32(_ALPHA) * bl
    res = jnp.where(m <= one, 0.0, total)
    res = jnp.where(mask_t > 0.5, res, jnp.float32(100.0))
    in_b = (ti_raw >= 0) & (ti_raw < n_real)
    res = jnp.where(in_b, res, one)
    out_ref[...] = res.astype(jnp.float32)


def kernel(scores, embeddings, target_idx, applicable_mask):
    del embeddings
    n = scores.shape[0]
    l_sub = -(-n // (_NS * _CH)) * _CH
    p_tot = l_sub * _NS
    ti_raw = jnp.asarray(target_idx, jnp.int32)
    ti = jnp.clip(ti_raw, 0, n - 1).astype(jnp.int32)

    scores_p = jnp.pad(scores.astype(jnp.float32), (0, p_tot - n),
                       constant_values=_NEG_INF)
    mask_p = jnp.pad(applicable_mask.astype(jnp.int32), (0, p_tot - n))
    ti_arr = jnp.full((_L,), ti, jnp.int32)

    sc = functools.partial(
        pl.kernel,
        out_type=(jax.ShapeDtypeStruct((_L,), jnp.float32),
                  jax.ShapeDtypeStruct((_L,), jnp.int32)),
        mesh=plsc.VectorSubcoreMesh(core_axis_name="c", subcore_axis_name="s",
                                    num_cores=1),
        compiler_params=pltpu.CompilerParams(needs_layout_passes=False),
        scratch_types=[
            pltpu.VMEM((l_sub,), jnp.int32),
            pltpu.VMEM((_CH,), jnp.float32),
            pltpu.VMEM((_CH,), jnp.int32),
            pltpu.VMEM((_L,), jnp.int32),
            pltpu.VMEM((_L,), jnp.float32),
            pltpu.VMEM((_L,), jnp.int32),
            pltpu.SMEM((_NWORDS,), jnp.int32),
        ],
    )(functools.partial(_sc_body, l_sub=l_sub, n_real=n))
    outf, outi = sc(scores_p, mask_p, ti_arr)

    res = pl.pallas_call(
        functools.partial(_finish_body, n_real=n),
        out_shape=jax.ShapeDtypeStruct((1, 1), jnp.float32),
    )(outf.reshape(1, _L), outi.reshape(1, _L), ti_raw.reshape(1, 1))
    return res[0, 0]


# speedup vs baseline: 5.5222x; 2.3748x over previous
import functools

import jax
import jax.numpy as jnp
from jax import lax
from jax.experimental import pallas as pl
from jax.experimental.pallas import tpu as pltpu
from jax.experimental.pallas import tpu_sc as plsc

_TEMP = 0.07
_MARGIN = 1.0
_ALPHA = 0.3
_NEG_INF = float("-inf")
_INT_MIN = -2147483648

_NS = 16
_L = 16
_CH = 8192

_W_ROUND = 0
_W_M = 32
_W_THI = 33
_W_TLO = 34
_W_MT = 35
_W_CM1 = 36
_W_CGT = 37
_MB_SMAX = 40
_MB_MAXI = 72
_MB_LI = 104
_MB_SEXP = 136
_MB_SM1 = 168
_MB_SGT = 200
_NWORDS = 240


def _xlane(v, op):
    idx = lax.iota(jnp.int32, _L)
    for sh in (1, 2, 4, 8):
        v = op(v, v.at[idx ^ sh].get(mode="promise_in_bounds"))
    return v[0]


def _to_skey(s):
    u = lax.bitcast_convert_type(s, jnp.int32)
    return jnp.where(u >= 0, u, jnp.int32(_INT_MIN) - u)


def _from_skey(key):
    u = jnp.where(key >= 0, key, jnp.int32(_INT_MIN) - key)
    return lax.bitcast_convert_type(u, jnp.float32)


def _halves(x_f32):
    b = lax.bitcast_convert_type(x_f32, jnp.int32)
    hi = lax.shift_right_logical(b, 16)
    lo = b & jnp.int32(0xFFFF)
    return hi, lo


def _sc_body(scores_hbm, mask_hbm, ti_hbm, outf_hbm, outi_hbm,
             a_ref, sbuf, mbuf, tibuf, obuf_f, obuf_i, sm,
             *, l_sub, n_real):
    nch = l_sub // _CH
    nstep = _CH // _L
    nvec = l_sub // _L
    wid = lax.axis_index("s")
    base = wid * l_sub
    neg_inf = jnp.float32(_NEG_INF)
    lanes = lax.iota(jnp.int32, _L)
    zero = jnp.float32(0.0)
    one_i = jnp.int32(1)

    pltpu.sync_copy(ti_hbm, tibuf)
    ti = tibuf[...][0]

    def _z(j, x):
        sm[j] = jnp.int32(0)
        return x

    lax.fori_loop(0, _NWORDS, _z, jnp.int32(0))
    plsc.subcore_barrier()

    def _poll(word, expect):
        def cond(v):
            return (v & jnp.int32(31)) != expect

        def body(v):
            return plsc.fetch_and_add(sm.at[word], jnp.int32(0),
                                      subcore_id=wid)

        v = lax.while_loop(cond, body, jnp.int32(0))
        return lax.shift_right_logical(v, 5)

    def chunk_body(c, carry):
        cnt, mxm, mxi, tac, mta, mxn = carry
        off = base + c * _CH
        pltpu.sync_copy(scores_hbm.at[pl.ds(off, _CH)], sbuf)
        pltpu.sync_copy(mask_hbm.at[pl.ds(off, _CH)], mbuf)

        def step(j, carry2):
            cnt, mxm, mxi, tac, mta, mxn = carry2
            p = j * _L
            s = sbuf[pl.ds(p, _L)]
            mk = mbuf[pl.ds(p, _L)]
            mkb = mk != 0
            gidx = off + p + lanes
            is_t = gidx == ti
            cnt = cnt + jnp.where(mkb, 1, 0).astype(jnp.int32)
            mxm = jnp.maximum(mxm, jnp.where(mkb, s, neg_inf))
            mxi = jnp.maximum(mxi, jnp.where(mkb, neg_inf, s))
            tac = jnp.maximum(tac, jnp.where(is_t, s, neg_inf))
            mta = jnp.maximum(mta, jnp.where(is_t, mk.astype(jnp.float32),
                                             neg_inf))
            negb = mkb & jnp.logical_not(is_t)
            mxn = jnp.maximum(mxn, jnp.where(negb, s, neg_inf))
            sk = _to_skey(s)
            a_ref[pl.ds(c * _CH + p, _L)] = jnp.where(negb, sk,
                                                      jnp.int32(_INT_MIN))
            return cnt, mxm, mxi, tac, mta, mxn

        return lax.fori_loop(0, nstep, step, carry)

    finit = jnp.full((_L,), neg_inf, jnp.float32)
    cnt, mxm, mxi, tac, mta, mxn = lax.fori_loop(
        0, nch, chunk_body,
        (jnp.zeros((_L,), jnp.int32), finit, finit, finit, finit, finit))

    m_loc = _xlane(cnt.astype(jnp.float32), jnp.add).astype(jnp.int32)
    smax_loc = _xlane(mxm, jnp.maximum)
    maxi_loc = _xlane(mxi, jnp.maximum)
    t_loc = _xlane(tac, jnp.maximum)
    mt_loc = _xlane(mta, jnp.maximum)
    li_loc = _xlane(mxn, jnp.maximum)

    inv_t = jnp.float32(1.0 / _TEMP)
    li_t = li_loc * inv_t

    def p2_step(j, sexp):
        sk = a_ref[pl.ds(j * _L, _L)]
        valid = sk != jnp.int32(_INT_MIN)
        s = _from_skey(sk)
        e = jnp.exp(s * inv_t - li_t)
        return sexp + jnp.where(valid, e, zero)

    sexp_loc = _xlane(
        lax.fori_loop(0, nvec, p2_step, jnp.zeros((_L,), jnp.float32)),
        jnp.add)
    sexp_loc = jnp.where(li_loc == neg_inf, zero, sexp_loc)

    owner = (ti >= base) & (ti < base + l_sub)
    thi, tlo = _halves(t_loc)
    smax_hi, smax_lo = _halves(smax_loc)
    maxi_hi, maxi_lo = _halves(maxi_loc)
    li_hi, li_lo = _halves(li_loc)
    se_hi, se_lo = _halves(sexp_loc)
    mt_int = jnp.where(mt_loc > jnp.float32(0.5), one_i, jnp.int32(0))
    zi = jnp.int32(0)

    def m1_send(j, x):
        tgt = j & jnp.int32(15)
        ws = lax.shift_right_logical(j, 4)
        word = jnp.where(ws == 0, _W_M, zi)
        val = jnp.where(ws == 0, m_loc, zi)
        word = jnp.where(ws == 1, _W_THI, word)
        val = jnp.where(ws == 1, jnp.where(owner, thi, zi), val)
        word = jnp.where(ws == 2, _W_TLO, word)
        val = jnp.where(ws == 2, jnp.where(owner, tlo, zi), val)
        word = jnp.where(ws == 3, _W_MT, word)
        val = jnp.where(ws == 3, jnp.where(owner, mt_int, zi), val)
        mb = ws - 4
        mtyp = lax.shift_right_logical(mb, 1)
        half = mb & one_i
        mb_base = jnp.where(mtyp == 0, _MB_SMAX, zi)
        mb_val_h = jnp.where(mtyp == 0, smax_hi, zi)
        mb_val_l = jnp.where(mtyp == 0, smax_lo, zi)
        mb_base = jnp.where(mtyp == 1, _MB_MAXI, mb_base)
        mb_val_h = jnp.where(mtyp == 1, maxi_hi, mb_val_h)
        mb_val_l = jnp.where(mtyp == 1, maxi_lo, mb_val_l)
        mb_base = jnp.where(mtyp == 2, _MB_LI, mb_base)
        mb_val_h = jnp.where(mtyp == 2, li_hi, mb_val_h)
        mb_val_l = jnp.where(mtyp == 2, li_lo, mb_val_l)
        mb_base = jnp.where(mtyp == 3, _MB_SEXP, mb_base)
        mb_val_h = jnp.where(mtyp == 3, se_hi, mb_val_h)
        mb_val_l = jnp.where(mtyp == 3, se_lo, mb_val_l)
        word = jnp.where(ws >= 4, mb_base + wid * 2 + half, word)
        val = jnp.where(ws >= 4, jnp.where(half == 0, mb_val_h, mb_val_l),
                        val)
        plsc.fetch_and_add(sm.at[word], val * 32 + 1, subcore_id=tgt)
        return x

    lax.fori_loop(0, 12 * _NS, m1_send, jnp.int32(0))

    def m1_poll(j, carry):
        sh_vec, hi0, lo0, hi1, lo1, hi2, lo2, hi3, lo3 = carry
        is_sh = j < 4
        writer = lax.shift_right_logical(j - 4, 3) & jnp.int32(15)
        rem = (j - 4) & jnp.int32(7)
        mtyp = lax.shift_right_logical(rem, 1)
        half = rem & one_i
        word = jnp.where(is_sh, _W_M + j, zi)
        mb_base = jnp.where(mtyp == 0, _MB_SMAX, zi)
        mb_base = jnp.where(mtyp == 1, _MB_MAXI, mb_base)
        mb_base = jnp.where(mtyp == 2, _MB_LI, mb_base)
        mb_base = jnp.where(mtyp == 3, _MB_SEXP, mb_base)
        word = jnp.where(is_sh, word, mb_base + writer * 2 + half)
        expect = jnp.where(is_sh, jnp.int32(_NS), one_i)
        val = _poll(word, expect)
        sh_vec = jnp.where(is_sh & (lanes == j), val, sh_vec)
        upd = jnp.logical_not(is_sh) & (lanes == writer)
        hi0 = jnp.where(upd & (mtyp == 0) & (half == 0), val, hi0)
        lo0 = jnp.where(upd & (mtyp == 0) & (half == 1), val, lo0)
        hi1 = jnp.where(upd & (mtyp == 1) & (half == 0), val, hi1)
        lo1 = jnp.where(upd & (mtyp == 1) & (half == 1), val, lo1)
        hi2 = jnp.where(upd & (mtyp == 2) & (half == 0), val, hi2)
        lo2 = jnp.where(upd & (mtyp == 2) & (half == 1), val, lo2)
        hi3 = jnp.where(upd & (mtyp == 3) & (half == 0), val, hi3)
        lo3 = jnp.where(upd & (mtyp == 3) & (half == 1), val, lo3)
        return sh_vec, hi0, lo0, hi1, lo1, hi2, lo2, hi3, lo3

    zv = jnp.zeros((_L,), jnp.int32)
    (sh_vec, hi0, lo0, hi1, lo1, hi2, lo2, hi3, lo3) = lax.fori_loop(
        0, 4 + 4 * _NS * 2, m1_poll, (zv,) * 9)

    def _vec_f32(hi, lo):
        return lax.bitcast_convert_type(
            lax.shift_left(hi, 16) | lo, jnp.float32)

    smax_vec = _vec_f32(hi0, lo0)
    maxi_vec = _vec_f32(hi1, lo1)
    li_vec = _vec_f32(hi2, lo2)
    sexp_vec = _vec_f32(hi3, lo3)

    m_g = sh_vec[0].astype(jnp.float32)
    t_bits = lax.shift_left(sh_vec[1], 16) | sh_vec[2]
    t_g = lax.bitcast_convert_type(t_bits, jnp.float32)
    maskt_g = sh_vec[3].astype(jnp.float32)
    smax_g = _xlane(smax_vec, jnp.maximum)
    maxinap_g = _xlane(maxi_vec, jnp.maximum)
    scale = jnp.exp((li_vec - smax_g) * inv_t)
    sumexp_neg = _xlane(jnp.where(li_vec == neg_inf, zero, sexp_vec * scale),
                        jnp.add)

    mi = m_g.astype(jnp.int32)
    k_i = jnp.maximum(one_i, (mi - 1) >> 1)

    def round_body(r, carry3):
        p_o = carry3
        b = 31 - r
        cand = p_o + jnp.left_shift(jnp.uint32(1), b.astype(jnp.uint32))
        cand_sk = lax.bitcast_convert_type(cand ^ jnp.uint32(0x80000000),
                                           jnp.int32)

        def cstep(j, ccnt):
            p = j * (4 * _L)
            for u in range(4):
                sk = a_ref[pl.ds(p + u * _L, _L)]
                ccnt = ccnt + jnp.where(sk >= cand_sk, 1, 0).astype(jnp.int32)
            return ccnt

        ccnt = lax.fori_loop(0, nvec // 4, cstep,
                             jnp.zeros((_L,), jnp.int32))
        c_loc = _xlane(ccnt.astype(jnp.float32), jnp.add).astype(jnp.int32)

        def r_send(j, x):
            plsc.fetch_and_add(sm.at[_W_ROUND + r], c_loc * 32 + 1,
                               subcore_id=j)
            return x

        lax.fori_loop(0, _NS, r_send, jnp.int32(0))
        c_g = _poll(_W_ROUND + r, jnp.int32(_NS))
        take = c_g >= k_i
        return jnp.where(take, cand, p_o)

    p_o = lax.fori_loop(0, 32, round_body, jnp.uint32(0))
    theta_key = lax.bitcast_convert_type(p_o ^ jnp.uint32(0x80000000),
                                         jnp.int32)

    t1key = _to_skey(jnp.broadcast_to(t_g - jnp.float32(_MARGIN), (_L,)))[0]

    def p4_step(j, carry4):
        cm1, sm1, cgt, sgt = carry4
        sk = a_ref[pl.ds(j * _L, _L)]
        valid = sk != jnp.int32(_INT_MIN)
        s = _from_skey(sk)
        gt1 = valid & (sk > t1key)
        gth = valid & (sk > theta_key)
        cm1 = cm1 + jnp.where(gt1, 1, 0).astype(jnp.int32)
        sm1 = sm1 + jnp.where(gt1, s, zero)
        cgt = cgt + jnp.where(gth, 1, 0).astype(jnp.int32)
        sgt = sgt + jnp.where(gth, s, zero)
        return cm1, sm1, cgt, sgt

    cm1, sm1, cgt, sgt = lax.fori_loop(
        0, nvec, p4_step,
        (zv, jnp.zeros((_L,), jnp.float32), zv,
         jnp.zeros((_L,), jnp.float32)))
    cm1_loc = _xlane(cm1.astype(jnp.float32), jnp.add).astype(jnp.int32)
    sm1_loc = _xlane(sm1, jnp.add)
    cgt_loc = _xlane(cgt.astype(jnp.float32), jnp.add).astype(jnp.int32)
    sgt_loc = _xlane(sgt, jnp.add)
    sm1_hi, sm1_lo = _halves(sm1_loc)
    sgt_hi, sgt_lo = _halves(sgt_loc)

    def f_send(j, x):
        tgt = j & jnp.int32(15)
        ws = lax.shift_right_logical(j, 4)
        word = jnp.where(ws == 0, _W_CM1, zi)
        val = jnp.where(ws == 0, cm1_loc, zi)
        word = jnp.where(ws == 1, _W_CGT, word)
        val = jnp.where(ws == 1, cgt_loc, val)
        word = jnp.where(ws == 2, _MB_SM1 + wid * 2, word)
        val = jnp.where(ws == 2, sm1_hi, val)
        word = jnp.where(ws == 3, _MB_SM1 + wid * 2 + 1, word)
        val = jnp.where(ws == 3, sm1_lo, val)
        word = jnp.where(ws == 4, _MB_SGT + wid * 2, word)
        val = jnp.where(ws == 4, sgt_hi, val)
        word = jnp.where(ws == 5, _MB_SGT + wid * 2 + 1, word)
        val = jnp.where(ws == 5, sgt_lo, val)
        plsc.fetch_and_add(sm.at[word], val * 32 + 1, subcore_id=tgt)
        return x

    lax.fori_loop(0, 6 * _NS, f_send, jnp.int32(0))

    def f_poll(j, carry):
        sh_vec, hiA, loA, hiB, loB = carry
        is_sh = j < 2
        writer = lax.shift_right_logical(j - 2, 2) & jnp.int32(15)
        rem = (j - 2) & jnp.int32(3)
        mtyp = lax.shift_right_logical(rem, 1)
        half = rem & one_i
        word = jnp.where(is_sh, _W_CM1 + j, zi)
        mb_base = jnp.where(mtyp == 0, _MB_SM1, jnp.int32(_MB_SGT))
        word = jnp.where(is_sh, word, mb_base + writer * 2 + half)
        expect = jnp.where(is_sh, jnp.int32(_NS), one_i)
        val = _poll(word, expect)
        sh_vec = jnp.where(is_sh & (lanes == j), val, sh_vec)
        upd = jnp.logical_not(is_sh) & (lanes == writer)
        hiA = jnp.where(upd & (mtyp == 0) & (half == 0), val, hiA)
        loA = jnp.where(upd & (mtyp == 0) & (half == 1), val, loA)
        hiB = jnp.where(upd & (mtyp == 1) & (half == 0), val, hiB)
        loB = jnp.where(upd & (mtyp == 1) & (half == 1), val, loB)
        return sh_vec, hiA, loA, hiB, loB

    (fsh, hiA, loA, hiB, loB) = lax.fori_loop(
        0, 2 + 2 * _NS * 2, f_poll, (zv,) * 5)
    cnt_m1 = fsh[0].astype(jnp.float32)
    cnt_gt = fsh[1].astype(jnp.float32)
    sum_m1 = _xlane(_vec_f32(hiA, loA), jnp.add)
    sum_gt = _xlane(_vec_f32(hiB, loB), jnp.add)

    @pl.when(wid == 0)
    def _():
        ov = jnp.zeros((_L,), jnp.float32)
        ov = jnp.where(lanes == 0, m_g, ov)
        ov = jnp.where(lanes == 1, smax_g, ov)
        ov = jnp.where(lanes == 2, maxinap_g, ov)
        ov = jnp.where(lanes == 3, t_g, ov)
        ov = jnp.where(lanes == 4, maskt_g, ov)
        ov = jnp.where(lanes == 5, sumexp_neg, ov)
        ov = jnp.where(lanes == 6, cnt_m1, ov)
        ov = jnp.where(lanes == 7, sum_m1, ov)
        ov = jnp.where(lanes == 8, cnt_gt, ov)
        ov = jnp.where(lanes == 9, sum_gt, ov)
        obuf_f[...] = ov
        obuf_i[...] = jnp.where(lanes == 0, theta_key, jnp.int32(0))
        pltpu.sync_copy(obuf_f, outf_hbm)
        pltpu.sync_copy(obuf_i, outi_hbm)


def _finish_body(f_ref, i_ref, ti_ref, out_ref, *, n_real):
    def g(i):
        return f_ref[0:1, i:i+1]

    m = g(0)
    smax = g(1)
    maxinap = g(2)
    t = g(3)
    mask_t = g(4)
    sumexp_neg = g(5)
    cnt_m1 = g(6)
    sum_m1 = g(7)
    cnt_gt = g(8)
    sum_gt = g(9)
    key = i_ref[0:1, 0:1]
    ti_raw = ti_ref[0:1, 0:1]

    temp = jnp.float32(_TEMP)
    one = jnp.float32(1.0)
    big_m = smax / temp
    lt = t / temp
    et = jnp.exp(lt - big_m)
    sumexp = sumexp_neg + jnp.where(mask_t > 0.5, et, 0.0)
    lse = jnp.log(sumexp) + big_m
    li = lse - lt
    p_t = et / sumexp
    fw = (one - p_t) * (one - p_t)
    li = fw * li

    mi = m.astype(jnp.int32)
    k_i = jnp.maximum(jnp.int32(1),
                      lax.shift_right_arithmetic(mi - 1, jnp.int32(1)))
    kf = k_i.astype(jnp.float32)

    u = jnp.where(key >= 0, key, jnp.int32(_INT_MIN) - key)
    theta = lax.bitcast_convert_type(u, jnp.float32)

    s_a = cnt_m1 * (one - t) + sum_m1
    s_b = cnt_gt * (one - t) + sum_gt + (kf - cnt_gt) * (one + theta - t)
    s_sel = jnp.where(cnt_m1 >= kf, s_b, s_a)
    hard = s_sel / kf

    bl = jnp.where(m < jnp.float32(n_real),
                   jnp.maximum(one + maxinap - t, 0.0), 0.0)

    total = li + jnp.float32(0.5) * hard + jnp.float32(_ALPHA) * bl
    res = jnp.where(m <= one, 0.0, total)
    res = jnp.where(mask_t > 0.5, res, jnp.float32(100.0))
    in_b = (ti_raw >= 0) & (ti_raw < n_real)
    res = jnp.where(in_b, res, one)
    out_ref[...] = res.astype(jnp.float32)


def kernel(scores, embeddings, target_idx, applicable_mask):
    del embeddings
    n = scores.shape[0]
    l_sub = -(-n // (_NS * _CH)) * _CH
    p_tot = l_sub * _NS
    ti_raw = jnp.asarray(target_idx, jnp.int32)
    ti = jnp.clip(ti_raw, 0, n - 1).astype(jnp.int32)

    scores_p = jnp.pad(scores.astype(jnp.float32), (0, p_tot - n),
                       constant_values=_NEG_INF)
    mask_p = jnp.pad(applicable_mask.astype(jnp.int32), (0, p_tot - n))
    ti_arr = jnp.full((_L,), ti, jnp.int32)

    sc = functools.partial(
        pl.kernel,
        out_type=(jax.ShapeDtypeStruct((_L,), jnp.float32),
                  jax.ShapeDtypeStruct((_L,), jnp.int32)),
        mesh=plsc.VectorSubcoreMesh(core_axis_name="c", subcore_axis_name="s",
                                    num_cores=1),
        compiler_params=pltpu.CompilerParams(needs_layout_passes=False),
        scratch_types=[
            pltpu.VMEM((l_sub,), jnp.int32),
            pltpu.VMEM((_CH,), jnp.float32),
            pltpu.VMEM((_CH,), jnp.int32),
            pltpu.VMEM((_L,), jnp.int32),
            pltpu.VMEM((_L,), jnp.float32),
            pltpu.VMEM((_L,), jnp.int32),
            pltpu.SMEM((_NWORDS,), jnp.int32),
        ],
    )(functools.partial(_sc_body, l_sub=l_sub, n_real=n))
    outf, outi = sc(scores_p, mask_p, ti_arr)

    res = pl.pallas_call(
        functools.partial(_finish_body, n_real=n),
        out_shape=jax.ShapeDtypeStruct((1, 1), jnp.float32),
    )(outf.reshape(1, _L), outi.reshape(1, _L), ti_raw.reshape(1, 1))
    return res[0, 0]
